# baseline (speedup 1.0000x reference)
import jax
import jax.numpy as jnp
from jax.experimental import pallas as pl
from jax.experimental.pallas import tpu as pltpu


def _gelu(x):
    return 0.5 * x * (1.0 + jax.lax.erf(x * (2.0 ** -0.5)))


def _gap_mlp_pair(rgb_f, t_f, fp_dw, fp_db, fp_uw, fp_ub):
    b, c, hw = rgb_f.shape
    inv = 1.0 / float(hw)
    db = fp_db[:, None]
    ub = fp_ub[:, None]

    def one(x, dw_ref, uw_ref, db_col, ub_col):
        gap = jnp.sum(x, axis=1, keepdims=True) * inv
        h1 = jax.lax.dot_general(dw_ref[...], gap, (((0,), (0,)), ((), ())),
                                 preferred_element_type=jnp.float32) + db_col
        h1 = _gelu(h1)
        y = jax.lax.dot_general(uw_ref[...], h1, (((0,), (0,)), ((), ())),
                                preferred_element_type=jnp.float32) + ub_col
        return y * jax.lax.rsqrt(jnp.sum(y * y, axis=0, keepdims=True))

    def kern(r_ref, t_ref, dw_ref, db_ref, uw_ref, ub_ref, or_ref, ot_ref):
        db_col = db_ref[...]
        ub_col = ub_ref[...]
        or_ref[...] = one(r_ref[0], dw_ref, uw_ref, db_col, ub_col)[None]
        ot_ref[...] = one(t_ref[0], dw_ref, uw_ref, db_col, ub_col)[None]

    o_r, o_t = pl.pallas_call(
        kern,
        out_shape=(jax.ShapeDtypeStruct((b, c, 1), jnp.float32),) * 2,
        grid=(b,),
        in_specs=[pl.BlockSpec((1, c, hw), lambda i: (i, 0, 0)),
                  pl.BlockSpec((1, c, hw), lambda i: (i, 0, 0)),
                  pl.BlockSpec((c, 2 * c), lambda i: (0, 0)),
                  pl.BlockSpec((2 * c, 1), lambda i: (0, 0)),
                  pl.BlockSpec((2 * c, c), lambda i: (0, 0)),
                  pl.BlockSpec((c, 1), lambda i: (0, 0))],
        out_specs=(pl.BlockSpec((1, c, 1), lambda i: (i, 0, 0)),
                   pl.BlockSpec((1, c, 1), lambda i: (i, 0, 0))),
        compiler_params=pltpu.CompilerParams(
            dimension_semantics=("parallel",),
            vmem_limit_bytes=48 << 20),
    )(rgb_f, t_f, fp_dw, db, fp_uw, ub)
    return o_r[:, :, 0], o_t[:, :, 0]


def _conv7_gates(rgb_f, t_f, cg_ext, taps, masks, dwb, edw, edb, euw, eub,
                 h, w):
    b, c, hw = rgb_f.shape
    c2 = 2 * c

    def kern(xr_ref, xt_ref, g_ref, tap_ref, m_ref, dwb_ref,
             edw_ref, edb_ref, euw_ref, eub_ref, ogr_ref, ogt_ref, xs_ref):
        g = g_ref[0]
        xg_r = (xr_ref[0] * g[:c]).astype(jnp.bfloat16)
        xg_t = (xt_ref[0] * g[c:]).astype(jnp.bfloat16)
        for dc in range(-3, 4):
            blk = (dc + 3) * c2
            if dc == 0:
                xs_ref[blk:blk + c] = xg_r
                xs_ref[blk + c:blk + c2] = xg_t
            else:
                sh = (-dc) % hw
                mrow = dc + 3 if dc < 0 else dc + 2
                cm = m_ref[mrow:mrow + 1, :].astype(jnp.bfloat16)
                xs_ref[blk:blk + c] = pltpu.roll(xg_r, sh, axis=1) * cm
                xs_ref[blk + c:blk + c2] = pltpu.roll(xg_t, sh, axis=1) * cm
        xs = xs_ref[...]
        acc = None
        for dr in range(-3, 4):
            part = jax.lax.dot_general(
                tap_ref[dr + 3], xs, (((1,), (0,)), ((), ())),
                preferred_element_type=jnp.float32)
            if dr == 0:
                acc = part if acc is None else acc + part
            else:
                sh = (-dr * w) % hw
                rm = m_ref[(9 + dr if dr < 0 else 8 + dr):(9 + dr if dr < 0 else 8 + dr) + 1, :]
                contrib = pltpu.roll(part, sh, axis=1) * rm
                acc = contrib if acc is None else acc + contrib
        gm = jnp.max(acc, axis=1, keepdims=True) + dwb_ref[...]
        h1 = _gelu(jax.lax.dot_general(
            edw_ref[...], gm, (((0,), (0,)), ((), ())),
            preferred_element_type=jnp.float32) + edb_ref[...])
        fg = jax.nn.sigmoid(jax.lax.dot_general(
            euw_ref[...], h1, (((0,), (0,)), ((), ())),
            preferred_element_type=jnp.float32) + eub_ref[...])
        cg = g_ref[0, :c]
        ogr_ref[...] = (cg * fg[:c] + (1.0 - cg))[None]
        ogt_ref[...] = (cg * fg[c:] + (1.0 - cg))[None]

    g_rgb, g_t = pl.pallas_call(
        kern,
        out_shape=(jax.ShapeDtypeStruct((b, c, 1), jnp.float32),) * 2,
        grid=(b,),
        in_specs=[
            pl.BlockSpec((1, c, hw), lambda i: (i, 0, 0)),
            pl.BlockSpec((1, c, hw), lambda i: (i, 0, 0)),
            pl.BlockSpec((1, c2, 1), lambda i: (i, 0, 0)),
            pl.BlockSpec((7, c2, 7 * c2), lambda i: (0, 0, 0)),
            pl.BlockSpec((16, hw), lambda i: (0, 0)),
            pl.BlockSpec((c2, 1), lambda i: (0, 0)),
            pl.BlockSpec((c2, c2 // 16), lambda i: (0, 0)),
            pl.BlockSpec((c2 // 16, 1), lambda i: (0, 0)),
            pl.BlockSpec((c2 // 16, c2), lambda i: (0, 0)),
            pl.BlockSpec((c2, 1), lambda i: (0, 0)),
        ],
        out_specs=(pl.BlockSpec((1, c, 1), lambda i: (i, 0, 0)),
                   pl.BlockSpec((1, c, 1), lambda i: (i, 0, 0))),
        scratch_shapes=[pltpu.VMEM((7 * c2, hw), jnp.bfloat16)],
        compiler_params=pltpu.CompilerParams(
            dimension_semantics=("arbitrary",),
            vmem_limit_bytes=56 << 20),
    )(rgb_f, t_f, cg_ext, taps, masks, dwb, edw, edb, euw, eub)
    return g_rgb, g_t


def _fuse(rgb_f, t_f, g_rgb, g_t, wr, br, wt, bt):
    b, c, hw = rgb_f.shape
    wr_row = wr.reshape(1, c).astype(jnp.float32)
    wt_row = wt.reshape(1, c).astype(jnp.float32)
    bdiff = (br - bt).reshape(1, 1).astype(jnp.float32)

    def kern(r_ref, t_ref, gr_ref, gt_ref, wr_ref, wt_ref, bd_ref,
             or_ref, ot_ref, of_ref):
        nr = r_ref[0] * gr_ref[0]
        nt = t_ref[0] * gt_ref[0]
        d = (jnp.dot(wr_ref[...], nr, preferred_element_type=jnp.float32)
             - jnp.dot(wt_ref[...], nt, preferred_element_type=jnp.float32)
             + bd_ref[0, 0])
        a = jax.nn.sigmoid(d)
        o_r = nr * a
        o_t = nt * (1.0 - a)
        or_ref[...] = o_r[None]
        ot_ref[...] = o_t[None]
        of_ref[...] = (o_r + o_t)[None]

    return pl.pallas_call(
        kern,
        out_shape=(jax.ShapeDtypeStruct((b, c, hw), jnp.float32),) * 3,
        grid=(b,),
        in_specs=[
            pl.BlockSpec((1, c, hw), lambda i: (i, 0, 0)),
            pl.BlockSpec((1, c, hw), lambda i: (i, 0, 0)),
            pl.BlockSpec((1, c, 1), lambda i: (i, 0, 0)),
            pl.BlockSpec((1, c, 1), lambda i: (i, 0, 0)),
            pl.BlockSpec((1, c), lambda i: (0, 0)),
            pl.BlockSpec((1, c), lambda i: (0, 0)),
            pl.BlockSpec(memory_space=pltpu.MemorySpace.SMEM),
        ],
        out_specs=(
            pl.BlockSpec((1, c, hw), lambda i: (i, 0, 0)),
            pl.BlockSpec((1, c, hw), lambda i: (i, 0, 0)),
            pl.BlockSpec((1, c, hw), lambda i: (i, 0, 0)),
        ),
        compiler_params=pltpu.CompilerParams(
            dimension_semantics=("parallel",),
            vmem_limit_bytes=48 << 20),
    )(rgb_f, t_f, g_rgb, g_t, wr_row, wt_row, bdiff)


def kernel(RGB, T, fp_dw, fp_db, fp_uw, fp_ub, dw_w, dw_b,
           ec_dw, ec_db, ec_uw, ec_ub, sr_w, sr_b, st_w, st_b):
    b, c, h, w = RGB.shape
    hw = h * w
    c2 = 2 * c
    rgb_f = RGB.reshape(b, c, hw)
    t_f = T.reshape(b, c, hw)

    rgb_y, t_y = _gap_mlp_pair(rgb_f, t_f, fp_dw, fp_db, fp_uw, fp_ub)

    rd = jnp.diagonal(rgb_y[:, :b])
    m = jax.nn.sigmoid(float(c) * rd[:, None] * t_y)
    cross_gate = m.T.reshape(b, c)

    w_flat = dw_w.reshape(c2, 2, 49)
    och = jnp.arange(c2)
    even = (och % 2 == 0)
    par = och + 1 - 2 * (och % 2)
    wA = jnp.where(even[:, None], w_flat[:, 0, :], w_flat[:, 1, :])
    wB = jnp.where(even[:, None], w_flat[par, 0, :], w_flat[par, 1, :])
    vB = wB[par]
    eye = (och[:, None] == och[None, :]).astype(jnp.float32)
    eyep = (par[:, None] == och[None, :]).astype(jnp.float32)
    blocks = (wA.T[:, :, None] * eye[None] +
              vB.T[:, :, None] * eyep[None])
    taps = (blocks.reshape(7, 7, c2, c2)
            .transpose(0, 2, 1, 3)
            .reshape(7, c2, 7 * c2)).astype(jnp.bfloat16)

    p = jnp.arange(hw)
    col = p % w
    row = p // w
    mask_rows = []
    for dc in (-3, -2, -1, 1, 2, 3):
        mask_rows.append(((col + dc >= 0) & (col + dc < w)))
    for dr in (-3, -2, -1, 1, 2, 3):
        mask_rows.append(((row + dr >= 0) & (row + dr < h)))
    masks = jnp.stack(mask_rows + [jnp.ones((hw,), jnp.bool_)] * 4
                      ).astype(jnp.float32)

    cg_ext = jnp.concatenate([cross_gate, cross_gate], axis=1)[:, :, None]
    g_rgb, g_t = _conv7_gates(rgb_f, t_f, cg_ext, taps, masks,
                              dw_b[:, None], ec_dw, ec_db[:, None],
                              ec_uw, ec_ub[:, None], h, w)

    o_rgb, o_t, o_fuse = _fuse(rgb_f, t_f, g_rgb, g_t, sr_w, sr_b, st_w, st_b)
    return (o_rgb.reshape(b, c, h, w),
            o_t.reshape(b, c, h, w),
            o_fuse.reshape(b, c, h, w))


# speedup vs baseline: 1.0497x; 1.0497x over previous
import jax
import jax.numpy as jnp
from jax.experimental import pallas as pl
from jax.experimental.pallas import tpu as pltpu


def _gelu(x):
    return 0.5 * x * (1.0 + jax.lax.erf(x * (2.0 ** -0.5)))


def _gap_mlp_pair(rgb_f, t_f, fp_dw, fp_db, fp_uw, fp_ub):
    b, c, hw = rgb_f.shape
    inv = 1.0 / float(hw)
    db = fp_db[:, None]
    ub = fp_ub[:, None]

    def one(x, dw_ref, uw_ref, db_col, ub_col):
        gap = jnp.sum(x, axis=1, keepdims=True) * inv
        h1 = jax.lax.dot_general(dw_ref[...], gap, (((0,), (0,)), ((), ())),
                                 preferred_element_type=jnp.float32) + db_col
        h1 = _gelu(h1)
        y = jax.lax.dot_general(uw_ref[...], h1, (((0,), (0,)), ((), ())),
                                preferred_element_type=jnp.float32) + ub_col
        return y * jax.lax.rsqrt(jnp.sum(y * y, axis=0, keepdims=True))

    def kern(r_ref, t_ref, dw_ref, db_ref, uw_ref, ub_ref, or_ref, ot_ref):
        db_col = db_ref[...]
        ub_col = ub_ref[...]
        or_ref[...] = one(r_ref[0], dw_ref, uw_ref, db_col, ub_col)[None]
        ot_ref[...] = one(t_ref[0], dw_ref, uw_ref, db_col, ub_col)[None]

    o_r, o_t = pl.pallas_call(
        kern,
        out_shape=(jax.ShapeDtypeStruct((b, c, 1), jnp.float32),) * 2,
        grid=(b,),
        in_specs=[pl.BlockSpec((1, c, hw), lambda i: (i, 0, 0)),
                  pl.BlockSpec((1, c, hw), lambda i: (i, 0, 0)),
                  pl.BlockSpec((c, 2 * c), lambda i: (0, 0)),
                  pl.BlockSpec((2 * c, 1), lambda i: (0, 0)),
                  pl.BlockSpec((2 * c, c), lambda i: (0, 0)),
                  pl.BlockSpec((c, 1), lambda i: (0, 0))],
        out_specs=(pl.BlockSpec((1, c, 1), lambda i: (i, 0, 0)),
                   pl.BlockSpec((1, c, 1), lambda i: (i, 0, 0))),
        compiler_params=pltpu.CompilerParams(
            dimension_semantics=("parallel",),
            vmem_limit_bytes=48 << 20),
    )(rgb_f, t_f, fp_dw, db, fp_uw, ub)
    return o_r[:, :, 0], o_t[:, :, 0]


def _conv7_fuse(rgb_f, t_f, cg_ext, taps, masks, dwb, edw, edb, euw, eub,
                sr_w, sr_b, st_w, st_b, h, w):
    b, c, hw = rgb_f.shape
    c2 = 2 * c
    wr_row = sr_w.reshape(1, c).astype(jnp.float32)
    wt_row = st_w.reshape(1, c).astype(jnp.float32)
    bdiff = (sr_b - st_b).reshape(1, 1).astype(jnp.float32)

    def kern(xr_ref, xt_ref, g_ref, tap_ref, m_ref, dwb_ref,
             edw_ref, edb_ref, euw_ref, eub_ref, wr_ref, wt_ref, bd_ref,
             or_ref, ot_ref, of_ref, xs_ref):
        g = g_ref[0]
        xg_r = (xr_ref[0] * g[:c]).astype(jnp.bfloat16)
        xg_t = (xt_ref[0] * g[c:]).astype(jnp.bfloat16)
        for dc in range(-3, 4):
            blk = (dc + 3) * c2
            if dc == 0:
                xs_ref[blk:blk + c] = xg_r
                xs_ref[blk + c:blk + c2] = xg_t
            else:
                sh = (-dc) % hw
                mrow = dc + 3 if dc < 0 else dc + 2
                cm = m_ref[mrow:mrow + 1, :].astype(jnp.bfloat16)
                xs_ref[blk:blk + c] = pltpu.roll(xg_r, sh, axis=1) * cm
                xs_ref[blk + c:blk + c2] = pltpu.roll(xg_t, sh, axis=1) * cm
        xs = xs_ref[...]
        acc = None
        for dr in range(-3, 4):
            part = jax.lax.dot_general(
                tap_ref[dr + 3], xs, (((1,), (0,)), ((), ())),
                preferred_element_type=jnp.float32)
            if dr == 0:
                acc = part if acc is None else acc + part
            else:
                sh = (-dr * w) % hw
                rm = m_ref[(9 + dr if dr < 0 else 8 + dr):(9 + dr if dr < 0 else 8 + dr) + 1, :]
                contrib = pltpu.roll(part, sh, axis=1) * rm
                acc = contrib if acc is None else acc + contrib
        gm = jnp.max(acc, axis=1, keepdims=True) + dwb_ref[...]
        h1 = _gelu(jax.lax.dot_general(
            edw_ref[...], gm, (((0,), (0,)), ((), ())),
            preferred_element_type=jnp.float32) + edb_ref[...])
        fg = jax.nn.sigmoid(jax.lax.dot_general(
            euw_ref[...], h1, (((0,), (0,)), ((), ())),
            preferred_element_type=jnp.float32) + eub_ref[...])
        cg = g_ref[0, :c]
        g_r = cg * fg[:c] + (1.0 - cg)
        g_t = cg * fg[c:] + (1.0 - cg)
        nr = xr_ref[0] * g_r
        nt = xt_ref[0] * g_t
        d = (jnp.dot(wr_ref[...], nr, preferred_element_type=jnp.float32)
             - jnp.dot(wt_ref[...], nt, preferred_element_type=jnp.float32)
             + bd_ref[0, 0])
        a = jax.nn.sigmoid(d)
        o_r = nr * a
        o_t = nt * (1.0 - a)
        or_ref[...] = o_r[None]
        ot_ref[...] = o_t[None]
        of_ref[...] = (o_r + o_t)[None]

    o_rgb, o_t, o_fuse = pl.pallas_call(
        kern,
        out_shape=(jax.ShapeDtypeStruct((b, c, hw), jnp.float32),) * 3,
        grid=(b,),
        in_specs=[
            pl.BlockSpec((1, c, hw), lambda i: (i, 0, 0)),
            pl.BlockSpec((1, c, hw), lambda i: (i, 0, 0)),
            pl.BlockSpec((1, c2, 1), lambda i: (i, 0, 0)),
            pl.BlockSpec((7, c2, 7 * c2), lambda i: (0, 0, 0)),
            pl.BlockSpec((16, hw), lambda i: (0, 0)),
            pl.BlockSpec((c2, 1), lambda i: (0, 0)),
            pl.BlockSpec((c2, c2 // 16), lambda i: (0, 0)),
            pl.BlockSpec((c2 // 16, 1), lambda i: (0, 0)),
            pl.BlockSpec((c2 // 16, c2), lambda i: (0, 0)),
            pl.BlockSpec((c2, 1), lambda i: (0, 0)),
            pl.BlockSpec((1, c), lambda i: (0, 0)),
            pl.BlockSpec((1, c), lambda i: (0, 0)),
            pl.BlockSpec(memory_space=pltpu.MemorySpace.SMEM),
        ],
        out_specs=(
            pl.BlockSpec((1, c, hw), lambda i: (i, 0, 0)),
            pl.BlockSpec((1, c, hw), lambda i: (i, 0, 0)),
            pl.BlockSpec((1, c, hw), lambda i: (i, 0, 0)),
        ),
        scratch_shapes=[pltpu.VMEM((7 * c2, hw), jnp.bfloat16)],
        compiler_params=pltpu.CompilerParams(
            dimension_semantics=("arbitrary",),
            vmem_limit_bytes=60 << 20),
    )(rgb_f, t_f, cg_ext, taps, masks, dwb, edw, edb, euw, eub,
      wr_row, wt_row, bdiff)
    return o_rgb, o_t, o_fuse


def kernel(RGB, T, fp_dw, fp_db, fp_uw, fp_ub, dw_w, dw_b,
           ec_dw, ec_db, ec_uw, ec_ub, sr_w, sr_b, st_w, st_b):
    b, c, h, w = RGB.shape
    hw = h * w
    c2 = 2 * c
    rgb_f = RGB.reshape(b, c, hw)
    t_f = T.reshape(b, c, hw)

    rgb_y, t_y = _gap_mlp_pair(rgb_f, t_f, fp_dw, fp_db, fp_uw, fp_ub)

    rd = jnp.diagonal(rgb_y[:, :b])
    m = jax.nn.sigmoid(float(c) * rd[:, None] * t_y)
    cross_gate = m.T.reshape(b, c)

    w_flat = dw_w.reshape(c2, 2, 49)
    och = jnp.arange(c2)
    even = (och % 2 == 0)
    par = och + 1 - 2 * (och % 2)
    wA = jnp.where(even[:, None], w_flat[:, 0, :], w_flat[:, 1, :])
    wB = jnp.where(even[:, None], w_flat[par, 0, :], w_flat[par, 1, :])
    vB = wB[par]
    eye = (och[:, None] == och[None, :]).astype(jnp.float32)
    eyep = (par[:, None] == och[None, :]).astype(jnp.float32)
    blocks = (wA.T[:, :, None] * eye[None] +
              vB.T[:, :, None] * eyep[None])
    taps = (blocks.reshape(7, 7, c2, c2)
            .transpose(0, 2, 1, 3)
            .reshape(7, c2, 7 * c2)).astype(jnp.bfloat16)

    p = jnp.arange(hw)
    col = p % w
    row = p // w
    mask_rows = []
    for dc in (-3, -2, -1, 1, 2, 3):
        mask_rows.append(((col + dc >= 0) & (col + dc < w)))
    for dr in (-3, -2, -1, 1, 2, 3):
        mask_rows.append(((row + dr >= 0) & (row + dr < h)))
    masks = jnp.stack(mask_rows + [jnp.ones((hw,), jnp.bool_)] * 4
                      ).astype(jnp.float32)

    cg_ext = jnp.concatenate([cross_gate, cross_gate], axis=1)[:, :, None]
    o_rgb, o_t, o_fuse = _conv7_fuse(rgb_f, t_f, cg_ext, taps, masks,
                                     dw_b[:, None], ec_dw, ec_db[:, None],
                                     ec_uw, ec_ub[:, None],
                                     sr_w, sr_b, st_w, st_b, h, w)
    return (o_rgb.reshape(b, c, h, w),
            o_t.reshape(b, c, h, w),
            o_fuse.reshape(b, c, h, w))
